# Initial kernel scaffold; baseline (speedup 1.0000x reference)
import jax
import jax.numpy as jnp
from jax.experimental import pallas as pl


def kernel(x_building, x_cable_group, x_transformer, edge_b2c_src, edge_b2c_dst, edge_c2t_src, edge_c2t_dst, edge_adj, enc_b_W, enc_b_b, enc_b_g, enc_b_beta, enc_c_W, enc_c_b, enc_c_g, enc_c_beta, enc_t_W, enc_t_b, enc_t_g, enc_t_beta, gat1_Ws, gat1_Wd, gat1_as, gat1_ad, gat1_b, gat2_Ws, gat2_Wd, gat2_as, gat2_ad, gat2_b, gcn_W, gcn_b):
    raise NotImplementedError("write your pallas kernel here")


# speedup vs baseline: 23.7412x; 23.7412x over previous
import functools

import jax
import jax.numpy as jnp
import numpy as np
from jax import lax
from jax.experimental import pallas as pl
from jax.experimental.pallas import tpu as pltpu
from jax.experimental.pallas import tpu_sc as plsc

HID, HEADS, HD = 64, 4, 16
N_B, N_C, N_T = 50000, 10000, 1000
BN_B, BN_C, BN_T = 51200, 10240, 1024
E1P, E2P, EAP = 53248, 12288, 802816
ROWS = 512

f32 = jnp.float32
bf16 = jnp.bfloat16
i32 = jnp.int32

_SEL16 = np.zeros((64, 16), np.float32)
for _h in range(4):
    _SEL16[_h * 16:(_h + 1) * 16, _h] = 1.0
_SEL8 = _SEL16[:, :8].copy()
_EXP80 = np.zeros((80, 64), np.float32)
for _h in range(4):
    _EXP80[64 + _h, _h * 16:(_h + 1) * 16] = 1.0
_PICK0 = np.zeros((4, 64), np.float32)
_PICK0[0, :] = 1.0


def _ln_relu(h, g, beta):
    m = jnp.mean(h, axis=-1, keepdims=True)
    v = jnp.mean((h - m) ** 2, axis=-1, keepdims=True)
    return jnp.maximum((h - m) * lax.rsqrt(v + 1e-5) * g + beta, 0.0)


def _a_b_body(x_ref, W_ref, b_ref, g_ref, be_ref, Ws_ref, af_ref, sel_ref,
              gW_ref, h_ref, hsx_ref, xw_ref):
    h = _ln_relu(jnp.dot(x_ref[...], W_ref[...],
                         preferred_element_type=f32) + b_ref[...],
                 g_ref[...], be_ref[...])
    h_ref[...] = h
    hs = jnp.dot(h, Ws_ref[...], preferred_element_type=f32)
    alx = jnp.dot(hs * af_ref[...], sel_ref[...], preferred_element_type=f32)
    hsx_ref[...] = jnp.concatenate([hs, alx], axis=1)
    xw_ref[...] = jnp.dot(h, gW_ref[...], preferred_element_type=f32)


def _a_d_body(x_ref, W_ref, b_ref, g_ref, be_ref, Wd_ref, adf_ref, sel_ref,
              h_ref, ald_ref):
    h = _ln_relu(jnp.dot(x_ref[...], W_ref[...],
                         preferred_element_type=f32) + b_ref[...],
                 g_ref[...], be_ref[...])
    h_ref[...] = h
    hd = jnp.dot(h, Wd_ref[...], preferred_element_type=f32)
    ald_ref[...] = jnp.dot(hd * adf_ref[...], sel_ref[...],
                           preferred_element_type=f32)


def _b_b_body(xw_ref, dis4_ref, pick_ref, lo_ref, hi_ref):
    dis64 = jnp.dot(dis4_ref[...], pick_ref[...], preferred_element_type=f32)
    xwp = xw_ref[...] * dis64
    lo_ref[...] = xwp[:, :32].astype(bf16)
    hi_ref[...] = xwp[:, 32:].astype(bf16)


def _b_c_body(h_ref, num_ref, b1_ref, exp_ref, Ws_ref, af_ref,
              sel_ref, hc2_ref, hsx_ref):
    x = num_ref[...]
    den64 = jnp.dot(x, exp_ref[...], preferred_element_type=f32)
    gat = x[:, :64] / (den64 + 1e-16) + b1_ref[...]
    hc2 = h_ref[...] + 0.5 * gat
    hc2_ref[...] = hc2
    hs = jnp.dot(hc2, Ws_ref[...], preferred_element_type=f32)
    alx = jnp.dot(hs * af_ref[...], sel_ref[...], preferred_element_type=f32)
    hsx_ref[...] = jnp.concatenate([hs, alx], axis=1)


def _c_b_body(h_ref, alo_ref, ahi_ref, xlo_ref, xhi_ref, dis4_ref, pick_ref,
              gb_ref, out_ref):
    acc = jnp.concatenate([alo_ref[...], ahi_ref[...]], axis=1).astype(f32)
    xwp = jnp.concatenate([xlo_ref[...], xhi_ref[...]], axis=1).astype(f32)
    dis64 = jnp.dot(dis4_ref[...], pick_ref[...], preferred_element_type=f32)
    out_ref[...] = h_ref[...] + 0.2 * (dis64 * (acc + xwp) + gb_ref[...])


def _c_t_body(h_ref, num_ref, exp_ref, b2_ref, out_ref):
    x = num_ref[...]
    den64 = jnp.dot(x, exp_ref[...], preferred_element_type=f32)
    out_ref[...] = h_ref[...] + 0.5 * (x[:, :64] / (den64 + 1e-16)
                                       + b2_ref[...])


def _row_spec(cols):
    return pl.BlockSpec((ROWS, cols), lambda i: (i, 0))


def _full_spec(r, c):
    return pl.BlockSpec((r, c), lambda i: (0, 0))


def _tc_call(body, n_rows, in_arrays, in_cols, out_cols):
    specs = []
    for a, c in zip(in_arrays, in_cols):
        if isinstance(c, tuple):
            specs.append(_full_spec(*c))
        else:
            specs.append(_row_spec(c))
    norm = [c if isinstance(c, tuple) else (c, f32) for c in out_cols]
    outs = [jax.ShapeDtypeStruct((n_rows, c), dt) for c, dt in norm]
    return pl.pallas_call(
        body,
        grid=(n_rows // ROWS,),
        in_specs=specs,
        out_specs=[_row_spec(c) for c, _ in norm],
        out_shape=outs,
    )(*in_arrays)


def _gat_edges(n_chunks, ebase, src_hbm, dst_hbm, hsx_hbm, aldx_hbm, rows,
               msg, alrows, sidx, didx, num_s, gsem, i16):
    mask4 = jnp.where(i16 < 4, 1.0, 0.0).astype(f32)

    def chunk(i, _):
        b = ebase + i * 128
        pltpu.sync_copy(src_hbm.at[pl.ds(b, 128)], sidx)
        pltpu.sync_copy(dst_hbm.at[pl.ds(b, 128)], didx)
        g1 = pltpu.async_copy(hsx_hbm.at[sidx], rows, gsem)
        g2 = pltpu.async_copy(aldx_hbm.at[didx], alrows, gsem)
        g1.wait()
        g2.wait()

        def grp(g, _):
            for e in range(16):
                ea = g * 16 + e
                av = rows[ea, pl.ds(64, 16)]
                ad = alrows[ea, pl.ds(0, 16)]
                ev = av + ad
                ev = jnp.where(ev >= 0.0, ev, 0.2 * ev)
                eev = jnp.exp(ev)
                msg[ea, pl.ds(64, 16)] = eev * mask4
                for h in range(4):
                    msg[ea, pl.ds(h * 16, 16)] = (
                        rows[ea, pl.ds(h * 16, 16)] * eev[h])
            return 0

        lax.fori_loop(0, 8, grp, 0)
        pltpu.sync_copy(msg, num_s.at[didx], add=True)
        return 0

    lax.fori_loop(0, n_chunks, chunk, 0)


_MESH = plsc.VectorSubcoreMesh(core_axis_name="c", subcore_axis_name="s")


_C_PT = BN_C // 16
_B_PT = BN_B // 16
_E1_PT = E1P // 16
_DEG_ROWS_PT = EAP // 128 // 16


@functools.partial(
    pl.kernel,
    out_type=[
        jax.ShapeDtypeStruct((BN_C, 80), f32),
        jax.ShapeDtypeStruct((BN_B * 4,), f32),
    ],
    mesh=_MESH,
    scratch_types=[
        pltpu.VMEM((128, 80), f32),
        pltpu.VMEM((128, 80), f32),
        pltpu.VMEM((128, 16), f32),
        pltpu.VMEM((128,), i32),
        pltpu.VMEM((128,), i32),
        pltpu.VMEM((8, 128), i32),
        pltpu.VMEM((128,), f32),
        pltpu.VMEM((640,), f32),
        pltpu.VMEM((2560,), f32),
        pltpu.VMEM_SHARED((BN_C, 80), f32),
        pltpu.VMEM_SHARED((BN_B,), f32),
        pltpu.SemaphoreType.DMA,
    ],
    compiler_params=pltpu.CompilerParams(needs_layout_passes=False, use_tc_tiling_on_sc=False),
)
def _sc1(hsx_hbm, aldx_hbm, src_hbm, dst_hbm, adst_hbm, z2d_hbm, z1d_hbm,
         num_hbm, dis4f_hbm,
         rows, msg, alrows, sidx, didx, didxb, onesv, degv, d4f,
         num_s, deg_s, gsem):
    cid = lax.axis_index("c")
    sid = lax.axis_index("s")
    i16 = lax.iota(i32, 16)
    z16 = jnp.zeros((16,), f32)

    @pl.when(cid == 1)
    def _():
        pltpu.sync_copy(z2d_hbm, num_s.at[pl.ds(sid * _C_PT, _C_PT), :])

    @pl.when(cid == 0)
    def _():
        pltpu.sync_copy(z1d_hbm, deg_s.at[pl.ds(sid * _B_PT, _B_PT)])
        for j in range(8):
            onesv[pl.ds(j * 16, 16)] = jnp.full((16,), 1.0, f32)

    plsc.subcore_barrier()

    @pl.when(cid == 1)
    def _():
        _gat_edges(_E1_PT // 128, sid * _E1_PT, src_hbm, dst_hbm, hsx_hbm,
                   aldx_hbm, rows, msg, alrows, sidx, didx, num_s, gsem, i16)

    @pl.when(cid == 0)
    def _():
        rbase = sid * _DEG_ROWS_PT

        def chunk(i, _):
            pltpu.sync_copy(adst_hbm.at[pl.ds(rbase + i * 8, 8), :], didxb)
            for j in range(8):
                pltpu.sync_copy(onesv, deg_s.at[didxb.at[j]], add=True)
            return 0

        lax.fori_loop(0, _DEG_ROWS_PT // 8, chunk, 0)

    plsc.subcore_barrier()

    @pl.when(cid == 1)
    def _():
        r0 = sid * _C_PT
        pltpu.sync_copy(num_s.at[pl.ds(r0, _C_PT), :],
                        num_hbm.at[pl.ds(r0, _C_PT), :])

    @pl.when(cid == 0)
    def _():
        def zf(j, _):
            off = pl.multiple_of(j * 16, 16)
            d4f[pl.ds(off, 16)] = z16
            return 0

        lax.fori_loop(0, 160, zf, 0)

        def rchunk(k, _):
            r0 = sid * _B_PT + k * 640
            pltpu.sync_copy(deg_s.at[pl.ds(r0, 640)], degv)

            def vr(j, _):
                off = pl.multiple_of(j * 16, 16)
                x = degv[pl.ds(off, 16)] + 1.0
                ii = plsc.bitcast(x, i32)
                ii = jnp.full((16,), 0x5F3759DF, i32) - (ii >> 1)
                y = plsc.bitcast(ii, f32)
                for _ in range(3):
                    y = y * (1.5 - 0.5 * x * y * y)
                plsc.store_scatter(d4f, [(j * 16 + i16) * 4], y)
                return 0

            lax.fori_loop(0, 40, vr, 0)
            pltpu.sync_copy(d4f, dis4f_hbm.at[pl.ds(r0 * 4, 2560)])
            return 0

        lax.fori_loop(0, 5, rchunk, 0)


_GCN_ROWS_PT = EAP // 128 // 16
_E2_PT = E2P // 16
_T_PT = BN_T // 16


@functools.partial(
    pl.kernel,
    out_type=[
        jax.ShapeDtypeStruct((BN_B, 32), bf16),
        jax.ShapeDtypeStruct((BN_B, 32), bf16),
        jax.ShapeDtypeStruct((BN_T, 80), f32),
    ],
    mesh=_MESH,
    scratch_types=[
        pltpu.VMEM((8, 128), i32),
        pltpu.VMEM((8, 128), i32),
        pltpu.VMEM((1024, 32), bf16),
        pltpu.VMEM((128, 80), f32),
        pltpu.VMEM((128, 80), f32),
        pltpu.VMEM((128, 16), f32),
        pltpu.VMEM((128,), i32),
        pltpu.VMEM((128,), i32),
        pltpu.VMEM_SHARED((BN_B, 32), bf16),
        pltpu.VMEM_SHARED((BN_T, 80), f32),
        pltpu.SemaphoreType.DMA,
    ],
    compiler_params=pltpu.CompilerParams(needs_layout_passes=False, use_tc_tiling_on_sc=False),
)
def _sc2(xlo_hbm, xhi_hbm, asrc_hbm, adst_hbm, hsx2_hbm, ald2x_hbm, src2_hbm,
         dst2_hbm, z32_hbm, z2dt_hbm,
         alo_hbm, ahi_hbm, num2_hbm,
         sidxb, didxb, rows32, rows, msg, alrows, sidx, didx,
         acc_s, num2_s, gsem):
    cid = lax.axis_index("c")
    sid = lax.axis_index("s")
    i16 = lax.iota(i32, 16)

    pltpu.sync_copy(z32_hbm, acc_s.at[pl.ds(sid * _B_PT, _B_PT), :])

    @pl.when(cid == 0)
    def _():
        pltpu.sync_copy(z2dt_hbm, num2_s.at[pl.ds(sid * _T_PT, _T_PT), :])

    plsc.subcore_barrier()

    def gcn_loop(xref):
        rbase = sid * _GCN_ROWS_PT

        def chunk(i, _):
            pltpu.sync_copy(asrc_hbm.at[pl.ds(rbase + i * 8, 8), :], sidxb)
            pltpu.sync_copy(adst_hbm.at[pl.ds(rbase + i * 8, 8), :], didxb)
            descs = []
            for j in range(8):
                descs.append(pltpu.async_copy(
                    xref.at[sidxb.at[j]],
                    rows32.at[pl.ds(j * 128, 128), :], gsem))
            for d in descs:
                d.wait()
            for j in range(8):
                pltpu.sync_copy(rows32.at[pl.ds(j * 128, 128), :],
                                acc_s.at[didxb.at[j]], add=True)
            return 0

        lax.fori_loop(0, _GCN_ROWS_PT // 8, chunk, 0)

    @pl.when(cid == 0)
    def _():
        gcn_loop(xlo_hbm)
        _gat_edges(_E2_PT // 128, sid * _E2_PT, src2_hbm, dst2_hbm, hsx2_hbm,
                   ald2x_hbm, rows, msg, alrows, sidx, didx, num2_s, gsem,
                   i16)

    @pl.when(cid == 1)
    def _():
        gcn_loop(xhi_hbm)

    plsc.subcore_barrier()

    r0 = sid * _B_PT

    @pl.when(cid == 0)
    def _():
        pltpu.sync_copy(acc_s.at[pl.ds(r0, _B_PT), :],
                        alo_hbm.at[pl.ds(r0, _B_PT), :])
        t0 = sid * _T_PT
        pltpu.sync_copy(num2_s.at[pl.ds(t0, _T_PT), :],
                        num2_hbm.at[pl.ds(t0, _T_PT), :])

    @pl.when(cid == 1)
    def _():
        pltpu.sync_copy(acc_s.at[pl.ds(r0, _B_PT), :],
                        ahi_hbm.at[pl.ds(r0, _B_PT), :])


def _pad_edges(e, n, fill):
    return jnp.concatenate([e, jnp.full((n - e.shape[0],), fill, e.dtype)])


def kernel(x_building, x_cable_group, x_transformer, edge_b2c_src,
           edge_b2c_dst, edge_c2t_src, edge_c2t_dst, edge_adj, enc_b_W,
           enc_b_b, enc_b_g, enc_b_beta, enc_c_W, enc_c_b, enc_c_g,
           enc_c_beta, enc_t_W, enc_t_b, enc_t_g, enc_t_beta, gat1_Ws,
           gat1_Wd, gat1_as, gat1_ad, gat1_b, gat2_Ws, gat2_Wd, gat2_as,
           gat2_ad, gat2_b, gcn_W, gcn_b):
    sel16 = jnp.asarray(_SEL16)
    sel8 = jnp.asarray(_SEL8)
    exp80 = jnp.asarray(_EXP80)
    pick0 = jnp.asarray(_PICK0)

    row = lambda a: a.reshape(1, -1)

    x_bp = jnp.pad(x_building, ((0, BN_B - N_B), (0, 32 - 17)))
    x_cp = jnp.pad(x_cable_group, ((0, BN_C - N_C), (0, 32 - 12)))
    x_tp = jnp.pad(x_transformer, ((0, BN_T - N_T), (0, 32 - 8)))
    Wb = jnp.pad(enc_b_W, ((0, 32 - 17), (0, 0)))
    Wc = jnp.pad(enc_c_W, ((0, 32 - 12), (0, 0)))
    Wt = jnp.pad(enc_t_W, ((0, 32 - 8), (0, 0)))

    s1 = _pad_edges(edge_b2c_src.astype(i32), E1P, 0)
    d1 = _pad_edges(edge_b2c_dst.astype(i32), E1P, 10100)
    s2 = _pad_edges(edge_c2t_src.astype(i32), E2P, 0)
    d2 = _pad_edges(edge_c2t_dst.astype(i32), E2P, 1016)
    sa = _pad_edges(edge_adj[0].astype(i32), EAP, 0).reshape(EAP // 128, 128)
    da = _pad_edges(edge_adj[1].astype(i32), EAP, 50100).reshape(EAP // 128, 128)

    h_b, hsx1, xw = _tc_call(
        _a_b_body, BN_B,
        [x_bp, Wb, row(enc_b_b), row(enc_b_g), row(enc_b_beta), gat1_Ws,
         row(gat1_as.reshape(-1)), sel16, gcn_W],
        [32, (32, 64), (1, 64), (1, 64), (1, 64), (64, 64), (1, 64),
         (64, 16), (64, 64)],
        [64, 80, 64])
    h_c, ald1 = _tc_call(
        _a_d_body, BN_C,
        [x_cp, Wc, row(enc_c_b), row(enc_c_g), row(enc_c_beta), gat1_Wd,
         row(gat1_ad.reshape(-1)), sel16],
        [32, (32, 64), (1, 64), (1, 64), (1, 64), (64, 64), (1, 64),
         (64, 16)],
        [64, 16])
    h_t, ald2 = _tc_call(
        _a_d_body, BN_T,
        [x_tp, Wt, row(enc_t_b), row(enc_t_g), row(enc_t_beta), gat2_Wd,
         row(gat2_ad.reshape(-1)), sel16],
        [32, (32, 64), (1, 64), (1, 64), (1, 64), (64, 64), (1, 64),
         (64, 16)],
        [64, 16])

    z2d = jnp.zeros((_C_PT, 80), f32)
    z1d = jnp.zeros((_B_PT,), f32)
    num1x, dis4f = _sc1(hsx1, ald1, s1, d1, da, z2d, z1d)
    dis4 = dis4f.reshape(BN_B, 4)

    xwp_lo, xwp_hi = _tc_call(
        _b_b_body, BN_B,
        [xw, dis4, pick0],
        [64, 4, (4, 64)],
        [(32, bf16), (32, bf16)])
    h_c2, hsx2 = _tc_call(
        _b_c_body, BN_C,
        [h_c, num1x, row(gat1_b), exp80, gat2_Ws,
         row(gat2_as.reshape(-1)), sel16],
        [64, 80, (1, 64), (80, 64), (64, 64), (1, 64), (64, 16)],
        [64, 80])

    z32 = jnp.zeros((_B_PT, 32), bf16)
    z2dt = jnp.zeros((_T_PT, 80), f32)
    acc_lo, acc_hi, num2x = _sc2(xwp_lo, xwp_hi, sa, da, hsx2,
                                 ald2, s2, d2, z32, z2dt)

    (out_b,) = _tc_call(
        _c_b_body, BN_B,
        [h_b, acc_lo, acc_hi, xwp_lo, xwp_hi, dis4, pick0, row(gcn_b)],
        [64, 32, 32, 32, 32, 4, (4, 64), (1, 64)],
        [64])
    (out_t,) = _tc_call(
        _c_t_body, BN_T,
        [h_t, num2x, exp80, row(gat2_b)],
        [64, 80, (80, 64), (1, 64)],
        [64])

    return jnp.concatenate([out_b[:N_B], h_c2[:N_C], out_t[:N_T]], axis=0)


# speedup vs baseline: 27.9990x; 1.1793x over previous
import functools

import jax
import jax.numpy as jnp
import numpy as np
from jax import lax
from jax.experimental import pallas as pl
from jax.experimental.pallas import tpu as pltpu
from jax.experimental.pallas import tpu_sc as plsc

HID, HEADS, HD = 64, 4, 16
N_B, N_C, N_T = 50000, 10000, 1000
BN_B, BN_C, BN_T = 51200, 10240, 1024
E1P, E2P, EAP = 53248, 12288, 802816
ROWS = 512

f32 = jnp.float32
bf16 = jnp.bfloat16
i32 = jnp.int32

_SEL16 = np.zeros((64, 16), np.float32)
for _h in range(4):
    _SEL16[_h * 16:(_h + 1) * 16, _h] = 1.0
_SEL8 = _SEL16[:, :8].copy()
_EXP80 = np.zeros((80, 64), np.float32)
for _h in range(4):
    _EXP80[64 + _h, _h * 16:(_h + 1) * 16] = 1.0
_PICK0 = np.zeros((4, 64), np.float32)
_PICK0[0, :] = 1.0


def _ln_relu(h, g, beta):
    m = jnp.mean(h, axis=-1, keepdims=True)
    v = jnp.mean((h - m) ** 2, axis=-1, keepdims=True)
    return jnp.maximum((h - m) * lax.rsqrt(v + 1e-5) * g + beta, 0.0)


def _a_b_body(x_ref, W_ref, b_ref, g_ref, be_ref, Ws_ref, af_ref, sel_ref,
              gW_ref, h_ref, hsx_ref, xw_ref):
    h = _ln_relu(jnp.dot(x_ref[...], W_ref[...],
                         preferred_element_type=f32) + b_ref[...],
                 g_ref[...], be_ref[...])
    h_ref[...] = h
    hs = jnp.dot(h, Ws_ref[...], preferred_element_type=f32)
    alx = jnp.dot(hs * af_ref[...], sel_ref[...], preferred_element_type=f32)
    hsx_ref[...] = jnp.concatenate([hs, alx], axis=1)
    xw_ref[...] = jnp.dot(h, gW_ref[...], preferred_element_type=f32)


def _a_d_body(x_ref, W_ref, b_ref, g_ref, be_ref, Wd_ref, adf_ref, sel_ref,
              h_ref, ald_ref):
    h = _ln_relu(jnp.dot(x_ref[...], W_ref[...],
                         preferred_element_type=f32) + b_ref[...],
                 g_ref[...], be_ref[...])
    h_ref[...] = h
    hd = jnp.dot(h, Wd_ref[...], preferred_element_type=f32)
    ald_ref[...] = jnp.dot(hd * adf_ref[...], sel_ref[...],
                           preferred_element_type=f32)


def _b_b_body(xw_ref, dis4_ref, pick_ref, lo_ref, hi_ref):
    dis64 = jnp.dot(dis4_ref[...], pick_ref[...], preferred_element_type=f32)
    xwp = xw_ref[...] * dis64
    lo_ref[...] = xwp[:, :32].astype(bf16)
    hi_ref[...] = xwp[:, 32:].astype(bf16)


def _b_c_body(h_ref, num_ref, b1_ref, exp_ref, Ws_ref, af_ref,
              sel_ref, hc2_ref, hsx_ref):
    x = num_ref[...]
    den64 = jnp.dot(x, exp_ref[...], preferred_element_type=f32)
    gat = x[:, :64] / (den64 + 1e-16) + b1_ref[...]
    hc2 = h_ref[...] + 0.5 * gat
    hc2_ref[...] = hc2
    hs = jnp.dot(hc2, Ws_ref[...], preferred_element_type=f32)
    alx = jnp.dot(hs * af_ref[...], sel_ref[...], preferred_element_type=f32)
    hsx_ref[...] = jnp.concatenate([hs, alx], axis=1)


def _c_b_body(h_ref, alo_ref, ahi_ref, xlo_ref, xhi_ref, dis4_ref, pick_ref,
              gb_ref, out_ref):
    acc = jnp.concatenate([alo_ref[...], ahi_ref[...]], axis=1).astype(f32)
    xwp = jnp.concatenate([xlo_ref[...], xhi_ref[...]], axis=1).astype(f32)
    dis64 = jnp.dot(dis4_ref[...], pick_ref[...], preferred_element_type=f32)
    out_ref[...] = h_ref[...] + 0.2 * (dis64 * (acc + xwp) + gb_ref[...])


def _c_t_body(h_ref, num_ref, exp_ref, b2_ref, out_ref):
    x = num_ref[...]
    den64 = jnp.dot(x, exp_ref[...], preferred_element_type=f32)
    out_ref[...] = h_ref[...] + 0.5 * (x[:, :64] / (den64 + 1e-16)
                                       + b2_ref[...])


def _row_spec(cols):
    return pl.BlockSpec((ROWS, cols), lambda i: (i, 0))


def _full_spec(r, c):
    return pl.BlockSpec((r, c), lambda i: (0, 0))


def _tc_call(body, n_rows, in_arrays, in_cols, out_cols):
    specs = []
    for a, c in zip(in_arrays, in_cols):
        if isinstance(c, tuple):
            specs.append(_full_spec(*c))
        else:
            specs.append(_row_spec(c))
    norm = [c if isinstance(c, tuple) else (c, f32) for c in out_cols]
    outs = [jax.ShapeDtypeStruct((n_rows, c), dt) for c, dt in norm]
    return pl.pallas_call(
        body,
        grid=(n_rows // ROWS,),
        in_specs=specs,
        out_specs=[_row_spec(c) for c, _ in norm],
        out_shape=outs,
    )(*in_arrays)


def _gat_edges(n_chunks, ebase, src_hbm, dst_hbm, hsx_hbm, aldx_hbm, rows,
               msg, alrows, sidx, didx, num_s, gsem, i16):
    mask4 = jnp.where(i16 < 4, 1.0, 0.0).astype(f32)

    def chunk(i, _):
        b = ebase + i * 128
        pltpu.sync_copy(src_hbm.at[pl.ds(b, 128)], sidx)
        pltpu.sync_copy(dst_hbm.at[pl.ds(b, 128)], didx)
        g1 = pltpu.async_copy(hsx_hbm.at[sidx], rows, gsem)
        g2 = pltpu.async_copy(aldx_hbm.at[didx], alrows, gsem)
        g1.wait()
        g2.wait()

        def grp(g, _):
            for e in range(16):
                ea = g * 16 + e
                av = rows[ea, pl.ds(64, 16)]
                ad = alrows[ea, pl.ds(0, 16)]
                ev = av + ad
                ev = jnp.where(ev >= 0.0, ev, 0.2 * ev)
                eev = jnp.exp(ev)
                msg[ea, pl.ds(64, 16)] = eev * mask4
                for h in range(4):
                    msg[ea, pl.ds(h * 16, 16)] = (
                        rows[ea, pl.ds(h * 16, 16)] * eev[h])
            return 0

        lax.fori_loop(0, 8, grp, 0)
        pltpu.sync_copy(msg, num_s.at[didx], add=True)
        return 0

    lax.fori_loop(0, n_chunks, chunk, 0)


def _gat_edges_pipe(n_chunks, ebase, src_hbm, dst_hbm, hsx_hbm, aldx_hbm,
                    rows0, rows1, alr0, alr1, msg, sidx0, didx0, sidx1,
                    didx1, num_s, semA, semB, i16):
    mask4 = jnp.where(i16 < 4, 1.0, 0.0).astype(f32)

    def load_idx(c, s, d):
        b = ebase + c * 128
        pltpu.sync_copy(src_hbm.at[pl.ds(b, 128)], s)
        pltpu.sync_copy(dst_hbm.at[pl.ds(b, 128)], d)

    def fire(s, d, rows, alr, sem):
        pltpu.async_copy(hsx_hbm.at[s], rows, sem)
        pltpu.async_copy(aldx_hbm.at[d], alr, sem)

    def wait_all(rows, alr, sem):
        pltpu.make_async_copy(hsx_hbm.at[pl.ds(0, 128), :], rows, sem).wait()
        pltpu.make_async_copy(aldx_hbm.at[pl.ds(0, 128), :], alr, sem).wait()

    def compute_scatter(rows, alr, didx):
        def grp(g, _):
            for e in range(16):
                ea = g * 16 + e
                av = rows[ea, pl.ds(64, 16)]
                ad = alr[ea, pl.ds(0, 16)]
                ev = av + ad
                ev = jnp.where(ev >= 0.0, ev, 0.2 * ev)
                eev = jnp.exp(ev)
                msg[ea, pl.ds(64, 16)] = eev * mask4
                for h in range(4):
                    msg[ea, pl.ds(h * 16, 16)] = (
                        rows[ea, pl.ds(h * 16, 16)] * eev[h])
            return 0

        lax.fori_loop(0, 8, grp, 0)
        pltpu.sync_copy(msg, num_s.at[didx], add=True)

    load_idx(0, sidx0, didx0)
    fire(sidx0, didx0, rows0, alr0, semA)

    def pair(k, _):
        c1 = 2 * k + 1
        load_idx(c1, sidx1, didx1)
        fire(sidx1, didx1, rows1, alr1, semB)
        wait_all(rows0, alr0, semA)
        compute_scatter(rows0, alr0, didx0)

        @pl.when(c1 + 1 < n_chunks)
        def _():
            load_idx(c1 + 1, sidx0, didx0)
            fire(sidx0, didx0, rows0, alr0, semA)

        wait_all(rows1, alr1, semB)
        compute_scatter(rows1, alr1, didx1)
        return 0

    lax.fori_loop(0, n_chunks // 2, pair, 0)


_MESH = plsc.VectorSubcoreMesh(core_axis_name="c", subcore_axis_name="s")


_C_PT = BN_C // 16
_B_PT = BN_B // 16
_E1_PT = E1P // 16
_DEG_ROWS_PT = EAP // 128 // 16


@functools.partial(
    pl.kernel,
    out_type=[
        jax.ShapeDtypeStruct((BN_C, 80), f32),
        jax.ShapeDtypeStruct((BN_B * 4,), f32),
    ],
    mesh=_MESH,
    scratch_types=[
        pltpu.VMEM((128, 80), f32),
        pltpu.VMEM((128, 80), f32),
        pltpu.VMEM((128, 80), f32),
        pltpu.VMEM((128, 16), f32),
        pltpu.VMEM((128, 16), f32),
        pltpu.VMEM((128,), i32),
        pltpu.VMEM((128,), i32),
        pltpu.VMEM((128,), i32),
        pltpu.VMEM((128,), i32),
        pltpu.VMEM((8, 128), i32),
        pltpu.VMEM((8, 128), i32),
        pltpu.VMEM((128,), f32),
        pltpu.VMEM((640,), f32),
        pltpu.VMEM((2560,), f32),
        pltpu.VMEM_SHARED((BN_C, 80), f32),
        pltpu.VMEM_SHARED((BN_B,), f32),
        pltpu.SemaphoreType.DMA,
        pltpu.SemaphoreType.DMA,
    ],
    compiler_params=pltpu.CompilerParams(needs_layout_passes=False, use_tc_tiling_on_sc=False),
)
def _sc1(hsx_hbm, aldx_hbm, src_hbm, dst_hbm, adst_hbm, z2d_hbm, z1d_hbm,
         num_hbm, dis4f_hbm,
         rows0, rows1, msg, alr0, alr1, sidx0, didx0, sidx1, didx1,
         didxb0, didxb1, onesv, degv, d4f,
         num_s, deg_s, gsemA, gsemB):
    cid = lax.axis_index("c")
    sid = lax.axis_index("s")
    i16 = lax.iota(i32, 16)
    z16 = jnp.zeros((16,), f32)

    @pl.when(cid == 1)
    def _():
        pltpu.sync_copy(z2d_hbm, num_s.at[pl.ds(sid * _C_PT, _C_PT), :])

    @pl.when(cid == 0)
    def _():
        pltpu.sync_copy(z1d_hbm, deg_s.at[pl.ds(sid * _B_PT, _B_PT)])
        for j in range(8):
            onesv[pl.ds(j * 16, 16)] = jnp.full((16,), 1.0, f32)

    plsc.subcore_barrier()

    @pl.when(cid == 1)
    def _():
        _gat_edges_pipe(_E1_PT // 128, sid * _E1_PT, src_hbm, dst_hbm,
                        hsx_hbm, aldx_hbm, rows0, rows1, alr0, alr1, msg,
                        sidx0, didx0, sidx1, didx1, num_s, gsemA, gsemB, i16)

    @pl.when(cid == 0)
    def _():
        rbase = sid * _DEG_ROWS_PT
        nch = _DEG_ROWS_PT // 8

        def load_idx(c, buf, sem):
            pltpu.async_copy(adst_hbm.at[pl.ds(rbase + c * 8, 8), :],
                             buf, sem)

        def wait_idx(buf, sem):
            pltpu.make_async_copy(adst_hbm.at[pl.ds(0, 8), :], buf,
                                  sem).wait()

        def scat(buf):
            for j in range(8):
                pltpu.sync_copy(onesv, deg_s.at[buf.at[j]], add=True)

        load_idx(0, didxb0, gsemA)

        def pair(k, _):
            c1 = 2 * k + 1
            load_idx(c1, didxb1, gsemB)
            wait_idx(didxb0, gsemA)
            scat(didxb0)

            @pl.when(c1 + 1 < nch)
            def _():
                load_idx(c1 + 1, didxb0, gsemA)

            wait_idx(didxb1, gsemB)
            scat(didxb1)
            return 0

        lax.fori_loop(0, nch // 2, pair, 0)
        wait_idx(didxb0, gsemA)
        scat(didxb0)

    plsc.subcore_barrier()

    @pl.when(cid == 1)
    def _():
        r0 = sid * _C_PT
        pltpu.sync_copy(num_s.at[pl.ds(r0, _C_PT), :],
                        num_hbm.at[pl.ds(r0, _C_PT), :])

    @pl.when(cid == 0)
    def _():
        def zf(j, _):
            off = pl.multiple_of(j * 16, 16)
            d4f[pl.ds(off, 16)] = z16
            return 0

        lax.fori_loop(0, 160, zf, 0)

        def rchunk(k, _):
            r0 = sid * _B_PT + k * 640
            pltpu.sync_copy(deg_s.at[pl.ds(r0, 640)], degv)

            def vr(j, _):
                off = pl.multiple_of(j * 16, 16)
                x = degv[pl.ds(off, 16)] + 1.0
                ii = plsc.bitcast(x, i32)
                ii = jnp.full((16,), 0x5F3759DF, i32) - (ii >> 1)
                y = plsc.bitcast(ii, f32)
                for _ in range(3):
                    y = y * (1.5 - 0.5 * x * y * y)
                plsc.store_scatter(d4f, [(j * 16 + i16) * 4], y)
                return 0

            lax.fori_loop(0, 40, vr, 0)
            pltpu.sync_copy(d4f, dis4f_hbm.at[pl.ds(r0 * 4, 2560)])
            return 0

        lax.fori_loop(0, 5, rchunk, 0)


_GCN_ROWS_PT = EAP // 128 // 16
_E2_PT = E2P // 16
_T_PT = BN_T // 16


@functools.partial(
    pl.kernel,
    out_type=[
        jax.ShapeDtypeStruct((BN_B, 32), bf16),
        jax.ShapeDtypeStruct((BN_B, 32), bf16),
        jax.ShapeDtypeStruct((BN_T, 80), f32),
    ],
    mesh=_MESH,
    scratch_types=[
        pltpu.VMEM((8, 128), i32),
        pltpu.VMEM((8, 128), i32),
        pltpu.VMEM((8, 128), i32),
        pltpu.VMEM((8, 128), i32),
        pltpu.VMEM((1024, 32), bf16),
        pltpu.VMEM((1024, 32), bf16),
        pltpu.VMEM((128, 80), f32),
        pltpu.VMEM((128, 80), f32),
        pltpu.VMEM((128, 16), f32),
        pltpu.VMEM((128,), i32),
        pltpu.VMEM((128,), i32),
        pltpu.VMEM_SHARED((BN_B, 32), bf16),
        pltpu.VMEM_SHARED((BN_T, 80), f32),
        pltpu.SemaphoreType.DMA,
        pltpu.SemaphoreType.DMA,
    ],
    compiler_params=pltpu.CompilerParams(needs_layout_passes=False, use_tc_tiling_on_sc=False),
)
def _sc2(xlo_hbm, xhi_hbm, asrc_hbm, adst_hbm, hsx2_hbm, ald2x_hbm, src2_hbm,
         dst2_hbm, z32_hbm, z2dt_hbm,
         alo_hbm, ahi_hbm, num2_hbm,
         sidxb0, didxb0, sidxb1, didxb1, rows32a, rows32b, rows, msg, alrows,
         sidx, didx, acc_s, num2_s, gsemA, gsemB):
    cid = lax.axis_index("c")
    sid = lax.axis_index("s")
    i16 = lax.iota(i32, 16)

    pltpu.sync_copy(z32_hbm, acc_s.at[pl.ds(sid * _B_PT, _B_PT), :])

    @pl.when(cid == 0)
    def _():
        pltpu.sync_copy(z2dt_hbm, num2_s.at[pl.ds(sid * _T_PT, _T_PT), :])

    plsc.subcore_barrier()

    def gcn_loop(xref):
        rbase = sid * _GCN_ROWS_PT
        nch = _GCN_ROWS_PT // 8

        def load_idx(c, ibs, ibd):
            pltpu.sync_copy(asrc_hbm.at[pl.ds(rbase + c * 8, 8), :], ibs)
            pltpu.sync_copy(adst_hbm.at[pl.ds(rbase + c * 8, 8), :], ibd)

        def fire(ibs, rb, sem):
            for j in range(8):
                pltpu.async_copy(xref.at[ibs.at[j]],
                                 rb.at[pl.ds(j * 128, 128), :], sem)

        def wait_all(rb, sem):
            pltpu.make_async_copy(xref.at[pl.ds(0, 1024), :], rb, sem).wait()

        def scat(rb, ibd):
            for j in range(8):
                pltpu.sync_copy(rb.at[pl.ds(j * 128, 128), :],
                                acc_s.at[ibd.at[j]], add=True)

        load_idx(0, sidxb0, didxb0)
        fire(sidxb0, rows32a, gsemA)

        def pair(k, _):
            c1 = 2 * k + 1
            load_idx(c1, sidxb1, didxb1)
            fire(sidxb1, rows32b, gsemB)
            wait_all(rows32a, gsemA)
            scat(rows32a, didxb0)

            @pl.when(c1 + 1 < nch)
            def _():
                load_idx(c1 + 1, sidxb0, didxb0)
                fire(sidxb0, rows32a, gsemA)

            wait_all(rows32b, gsemB)
            scat(rows32b, didxb1)
            return 0

        lax.fori_loop(0, nch // 2, pair, 0)
        wait_all(rows32a, gsemA)
        scat(rows32a, didxb0)

    @pl.when(cid == 0)
    def _():
        gcn_loop(xlo_hbm)
        _gat_edges(_E2_PT // 128, sid * _E2_PT, src2_hbm, dst2_hbm, hsx2_hbm,
                   ald2x_hbm, rows, msg, alrows, sidx, didx, num2_s, gsemA,
                   i16)

    @pl.when(cid == 1)
    def _():
        gcn_loop(xhi_hbm)

    plsc.subcore_barrier()

    r0 = sid * _B_PT

    @pl.when(cid == 0)
    def _():
        pltpu.sync_copy(acc_s.at[pl.ds(r0, _B_PT), :],
                        alo_hbm.at[pl.ds(r0, _B_PT), :])
        t0 = sid * _T_PT
        pltpu.sync_copy(num2_s.at[pl.ds(t0, _T_PT), :],
                        num2_hbm.at[pl.ds(t0, _T_PT), :])

    @pl.when(cid == 1)
    def _():
        pltpu.sync_copy(acc_s.at[pl.ds(r0, _B_PT), :],
                        ahi_hbm.at[pl.ds(r0, _B_PT), :])


def _pad_edges(e, n, fill):
    return jnp.concatenate([e, jnp.full((n - e.shape[0],), fill, e.dtype)])


def kernel(x_building, x_cable_group, x_transformer, edge_b2c_src,
           edge_b2c_dst, edge_c2t_src, edge_c2t_dst, edge_adj, enc_b_W,
           enc_b_b, enc_b_g, enc_b_beta, enc_c_W, enc_c_b, enc_c_g,
           enc_c_beta, enc_t_W, enc_t_b, enc_t_g, enc_t_beta, gat1_Ws,
           gat1_Wd, gat1_as, gat1_ad, gat1_b, gat2_Ws, gat2_Wd, gat2_as,
           gat2_ad, gat2_b, gcn_W, gcn_b):
    sel16 = jnp.asarray(_SEL16)
    sel8 = jnp.asarray(_SEL8)
    exp80 = jnp.asarray(_EXP80)
    pick0 = jnp.asarray(_PICK0)

    row = lambda a: a.reshape(1, -1)

    x_bp = jnp.pad(x_building, ((0, BN_B - N_B), (0, 32 - 17)))
    x_cp = jnp.pad(x_cable_group, ((0, BN_C - N_C), (0, 32 - 12)))
    x_tp = jnp.pad(x_transformer, ((0, BN_T - N_T), (0, 32 - 8)))
    Wb = jnp.pad(enc_b_W, ((0, 32 - 17), (0, 0)))
    Wc = jnp.pad(enc_c_W, ((0, 32 - 12), (0, 0)))
    Wt = jnp.pad(enc_t_W, ((0, 32 - 8), (0, 0)))

    s1 = _pad_edges(edge_b2c_src.astype(i32), E1P, 0)
    d1 = _pad_edges(edge_b2c_dst.astype(i32), E1P, 10100)
    s2 = _pad_edges(edge_c2t_src.astype(i32), E2P, 0)
    d2 = _pad_edges(edge_c2t_dst.astype(i32), E2P, 1016)
    sa = _pad_edges(edge_adj[0].astype(i32), EAP, 0).reshape(EAP // 128, 128)
    da = _pad_edges(edge_adj[1].astype(i32), EAP, 50100).reshape(EAP // 128, 128)

    h_b, hsx1, xw = _tc_call(
        _a_b_body, BN_B,
        [x_bp, Wb, row(enc_b_b), row(enc_b_g), row(enc_b_beta), gat1_Ws,
         row(gat1_as.reshape(-1)), sel16, gcn_W],
        [32, (32, 64), (1, 64), (1, 64), (1, 64), (64, 64), (1, 64),
         (64, 16), (64, 64)],
        [64, 80, 64])
    h_c, ald1 = _tc_call(
        _a_d_body, BN_C,
        [x_cp, Wc, row(enc_c_b), row(enc_c_g), row(enc_c_beta), gat1_Wd,
         row(gat1_ad.reshape(-1)), sel16],
        [32, (32, 64), (1, 64), (1, 64), (1, 64), (64, 64), (1, 64),
         (64, 16)],
        [64, 16])
    h_t, ald2 = _tc_call(
        _a_d_body, BN_T,
        [x_tp, Wt, row(enc_t_b), row(enc_t_g), row(enc_t_beta), gat2_Wd,
         row(gat2_ad.reshape(-1)), sel16],
        [32, (32, 64), (1, 64), (1, 64), (1, 64), (64, 64), (1, 64),
         (64, 16)],
        [64, 16])

    z2d = jnp.zeros((_C_PT, 80), f32)
    z1d = jnp.zeros((_B_PT,), f32)
    num1x, dis4f = _sc1(hsx1, ald1, s1, d1, da, z2d, z1d)
    dis4 = dis4f.reshape(BN_B, 4)

    xwp_lo, xwp_hi = _tc_call(
        _b_b_body, BN_B,
        [xw, dis4, pick0],
        [64, 4, (4, 64)],
        [(32, bf16), (32, bf16)])
    h_c2, hsx2 = _tc_call(
        _b_c_body, BN_C,
        [h_c, num1x, row(gat1_b), exp80, gat2_Ws,
         row(gat2_as.reshape(-1)), sel16],
        [64, 80, (1, 64), (80, 64), (64, 64), (1, 64), (64, 16)],
        [64, 80])

    z32 = jnp.zeros((_B_PT, 32), bf16)
    z2dt = jnp.zeros((_T_PT, 80), f32)
    acc_lo, acc_hi, num2x = _sc2(xwp_lo, xwp_hi, sa, da, hsx2,
                                 ald2, s2, d2, z32, z2dt)

    (out_b,) = _tc_call(
        _c_b_body, BN_B,
        [h_b, acc_lo, acc_hi, xwp_lo, xwp_hi, dis4, pick0, row(gcn_b)],
        [64, 32, 32, 32, 32, 4, (4, 64), (1, 64)],
        [64])
    (out_t,) = _tc_call(
        _c_t_body, BN_T,
        [h_t, num2x, exp80, row(gat2_b)],
        [64, 80, (80, 64), (1, 64)],
        [64])

    return jnp.concatenate([out_b[:N_B], h_c2[:N_C], out_t[:N_T]], axis=0)
